# baseline (speedup 1.0000x reference)
import functools

import jax
import jax.numpy as jnp
from jax import lax
from jax.experimental import pallas as pl
from jax.experimental.pallas import tpu as pltpu
from jax.experimental.pallas import tpu_sc as plsc

B = 16384
F = 26
D = 32
NUM_CORES = 2
NUM_SUBCORES = 16
NW = NUM_CORES * NUM_SUBCORES
BW = B // NW
NBLK = BW // 128

_mesh = plsc.VectorSubcoreMesh(core_axis_name="c", subcore_axis_name="s")


@functools.partial(
    pl.kernel,
    mesh=_mesh,
    out_type=jax.ShapeDtypeStruct((F, D // 8, B // 128, 8, 128), jnp.float32),
    scratch_types=[
        pltpu.VMEM((4, NBLK, 8, 128), jnp.int32),
        pltpu.VMEM((BW, D), jnp.float32),
        pltpu.VMEM((BW, D), jnp.float32),
        pltpu.VMEM((D, BW), jnp.float32),
        pltpu.VMEM((D, BW), jnp.float32),
        pltpu.SemaphoreType.DMA,
        pltpu.SemaphoreType.DMA,
        pltpu.SemaphoreType.DMA,
        pltpu.SemaphoreType.DMA,
        pltpu.SemaphoreType.DMA,
    ],
    compiler_params=pltpu.CompilerParams(
        use_tc_tiling_on_sc=False, needs_layout_passes=False),
)
def _gather_sc(idx_hbm, table_hbm, y_hbm,
               idx_all, rows0, rows1, bufT0, bufT1,
               sem_i, sg0, sg1, so0, so1):
    wid = lax.axis_index("s") * NUM_CORES + lax.axis_index("c")
    b0 = wid * BW
    blk0 = wid * NBLK
    rows = (rows0, rows1)
    bufT = (bufT0, bufT1)
    sem_g = (sg0, sg1)
    sem_o = (so0, so1)

    pltpu.sync_copy(idx_hbm.at[:, pl.ds(blk0, NBLK)], idx_all)

    lanes = lax.broadcasted_iota(jnp.int32, (16,), 0)
    pat = [(lanes + s) & 15 for s in range(16)]

    def transpose(rows_v, bufT_v):
        def tbody(bb, carry):
            b_vec = lanes + bb * 16
            for h in range(2):
                for s in range(16):
                    d_vec = pat[s] if h == 0 else pat[s] + 16
                    v = plsc.load_gather(rows_v, [b_vec, d_vec])
                    plsc.store_scatter(bufT_v, [d_vec, b_vec], v)
            return carry

        lax.fori_loop(0, BW // 16, tbody, 0)

    def emit_tiles(f, bufT_v, sem):
        def wbody(t, carry):
            db = t // NBLK
            j = t % NBLK
            pltpu.async_copy(
                bufT_v.at[pl.ds(db * 8, 8), pl.ds(j * 128, 128)],
                y_hbm.at[f, db, blk0 + j],
                sem,
            )
            return carry

        lax.fori_loop(0, (D // 8) * NBLK, wbody, 0)

    def sem_wait(buf_v, sem):
        pltpu.make_async_copy(table_hbm.at[pl.ds(0, BW)], buf_v, sem).wait()

    def enqueue_gather(f, p):
        fb = f // 8
        fi = f % 8
        for j in range(NBLK):
            pltpu.async_copy(
                table_hbm.at[idx_all.at[fb, j, fi]],
                rows[p].at[pl.ds(j * 128, 128)],
                sem_g[p],
            )

    enqueue_gather(0, 0)

    def fbody(t, carry):
        f0 = 2 * t
        enqueue_gather(f0 + 1, 1)
        sem_wait(rows[0], sem_g[0])

        @pl.when(t >= 1)
        def _():
            sem_wait(rows[0], sem_o[0])

        transpose(rows[0], bufT[0])
        emit_tiles(f0, bufT[0], sem_o[0])

        @pl.when(t + 1 < F // 2)
        def _():
            enqueue_gather(f0 + 2, 0)

        sem_wait(rows[1], sem_g[1])

        @pl.when(t >= 1)
        def _():
            sem_wait(rows[1], sem_o[1])

        transpose(rows[1], bufT[1])
        emit_tiles(f0 + 1, bufT[1], sem_o[1])
        return carry

    lax.fori_loop(0, F // 2, fbody, 0)
    sem_wait(rows[0], sem_o[0])
    sem_wait(rows[1], sem_o[1])


def kernel(indices, table):
    idx_t = jnp.transpose(indices).astype(jnp.int32)
    idx32 = jnp.concatenate(
        [idx_t, jnp.zeros((32 - F, B), jnp.int32)], axis=0)
    idx4 = idx32.reshape(4, 8, B // 128, 128).transpose(0, 2, 1, 3)
    y = _gather_sc(idx4, table)
    return y.transpose(2, 4, 0, 1, 3).reshape(B, F, D)
